# baseline (speedup 1.0000x reference)
import jax
import jax.numpy as jnp
from jax.experimental import pallas as pl
from jax.experimental.pallas import tpu as pltpu

_BN = 4096


def _embed_kernel(embs_t_ref, ids_t_ref, out_ref):
    e = embs_t_ref[...]
    norm = jnp.sqrt(jnp.sum(e * e, axis=0, keepdims=True))
    normed = e / jnp.maximum(norm, 1e-12)
    out_ref[...] = jnp.dot(
        normed, ids_t_ref[...], preferred_element_type=jnp.float32
    )


def kernel(ids, embs):
    b, v = ids.shape
    _, d = embs.shape
    ids_t = ids.T
    embs_t = embs.T
    out_t = pl.pallas_call(
        _embed_kernel,
        grid=(b // _BN,),
        in_specs=[
            pl.BlockSpec((d, v), lambda i: (0, 0)),
            pl.BlockSpec((v, _BN), lambda i: (0, i)),
        ],
        out_specs=pl.BlockSpec((d, _BN), lambda i: (0, i)),
        out_shape=jax.ShapeDtypeStruct((d, b), jnp.float32),
        compiler_params=pltpu.CompilerParams(
            dimension_semantics=("arbitrary",)
        ),
    )(embs_t, ids_t)
    return out_t.T


# speedup vs baseline: 1.0153x; 1.0153x over previous
import jax
import jax.numpy as jnp
from jax.experimental import pallas as pl
from jax.experimental.pallas import tpu as pltpu

_BN = 1024


def _embed_kernel(embs_t_ref, ids_t_ref, out_ref):
    e = embs_t_ref[...]
    norm = jnp.sqrt(jnp.sum(e * e, axis=0, keepdims=True))
    normed = e / jnp.maximum(norm, 1e-12)
    out_ref[...] = jnp.dot(
        normed, ids_t_ref[...], preferred_element_type=jnp.float32
    )


def kernel(ids, embs):
    b, v = ids.shape
    _, d = embs.shape
    ids_t = ids.T
    embs_t = embs.T
    out_t = pl.pallas_call(
        _embed_kernel,
        grid=(b // _BN,),
        in_specs=[
            pl.BlockSpec((d, v), lambda i: (0, 0)),
            pl.BlockSpec((v, _BN), lambda i: (0, i)),
        ],
        out_specs=pl.BlockSpec((d, _BN), lambda i: (0, i)),
        out_shape=jax.ShapeDtypeStruct((d, b), jnp.float32),
        compiler_params=pltpu.CompilerParams(
            dimension_semantics=("arbitrary",)
        ),
    )(embs_t, ids_t)
    return out_t.T


# speedup vs baseline: 1.0228x; 1.0074x over previous
import jax
import jax.numpy as jnp
from jax.experimental import pallas as pl
from jax.experimental.pallas import tpu as pltpu

_BN = 2048
_NOPS = 2


def _embed_kernel(*refs):
    embs_t_ref = refs[0]
    ids_refs = refs[1 : 1 + _NOPS]
    out_ref = refs[1 + _NOPS]
    e = embs_t_ref[...]
    norm = jnp.sqrt(jnp.sum(e * e, axis=0, keepdims=True))
    normed = e / jnp.maximum(norm, 1e-12)
    for j in range(_NOPS):
        out_ref[:, j * _BN : (j + 1) * _BN] = jnp.dot(
            normed, ids_refs[j][...], preferred_element_type=jnp.float32
        )


def kernel(ids, embs):
    b, v = ids.shape
    _, d = embs.shape
    ids_t = ids.T
    embs_t = embs.T
    cols_per_step = _BN * _NOPS
    in_specs = [pl.BlockSpec((d, v), lambda i: (0, 0))]
    in_specs += [
        pl.BlockSpec((v, _BN), lambda i, j=j: (0, i * _NOPS + j))
        for j in range(_NOPS)
    ]
    out_t = pl.pallas_call(
        _embed_kernel,
        grid=(b // cols_per_step,),
        in_specs=in_specs,
        out_specs=pl.BlockSpec((d, cols_per_step), lambda i: (0, i)),
        out_shape=jax.ShapeDtypeStruct((d, b), jnp.float32),
        compiler_params=pltpu.CompilerParams(
            dimension_semantics=("arbitrary",)
        ),
    )(embs_t, *([ids_t] * _NOPS))
    return out_t.T


# speedup vs baseline: 1.0398x; 1.0166x over previous
import jax
import jax.numpy as jnp
from jax.experimental import pallas as pl
from jax.experimental.pallas import tpu as pltpu

_BN = 2048


def _embed_kernel(embs_t_ref, ids_t_ref, out_ref):
    e = embs_t_ref[...]
    norm = jnp.sqrt(jnp.sum(e * e, axis=0, keepdims=True))
    normed = e / jnp.maximum(norm, 1e-12)
    out_ref[...] = jnp.dot(
        normed, ids_t_ref[...], preferred_element_type=jnp.float32
    )


def kernel(ids, embs):
    b, v = ids.shape
    _, d = embs.shape
    ids_t = ids.T
    embs_t = embs.T
    out_t = pl.pallas_call(
        _embed_kernel,
        grid=(b // _BN,),
        in_specs=[
            pl.BlockSpec((d, v), lambda i: (0, 0)),
            pl.BlockSpec((v, _BN), lambda i: (0, i)),
        ],
        out_specs=pl.BlockSpec((d, _BN), lambda i: (0, i)),
        out_shape=jax.ShapeDtypeStruct((d, b), jnp.float32),
        compiler_params=pltpu.CompilerParams(
            dimension_semantics=("arbitrary",)
        ),
    )(embs_t, ids_t)
    return out_t.T


# speedup vs baseline: 1.0554x; 1.0150x over previous
import jax
import jax.numpy as jnp
from jax.experimental import pallas as pl
from jax.experimental.pallas import tpu as pltpu

_BN = 2048


def _embed_kernel(embs_t_ref, ids_t_ref, out_ref):
    e = embs_t_ref[...]
    norm = jnp.sqrt(jnp.sum(e * e, axis=0, keepdims=True))
    normed = e / jnp.maximum(norm, 1e-12)
    out_ref[...] = jnp.dot(
        normed, ids_t_ref[...], preferred_element_type=jnp.float32
    )


def kernel(ids, embs):
    b, v = ids.shape
    _, d = embs.shape
    ids_t = ids.T
    embs_t = embs.T
    out_t = pl.pallas_call(
        _embed_kernel,
        grid=(b // _BN,),
        in_specs=[
            pl.BlockSpec((d, v), lambda i: (0, 0)),
            pl.BlockSpec((v, _BN), lambda i: (0, i)),
        ],
        out_specs=pl.BlockSpec((d, _BN), lambda i: (0, i)),
        out_shape=jax.ShapeDtypeStruct((d, b), jnp.float32),
        compiler_params=pltpu.CompilerParams(
            dimension_semantics=("parallel",)
        ),
    )(embs_t, ids_t)
    return out_t.T
